# Initial kernel scaffold; baseline (speedup 1.0000x reference)
import jax
import jax.numpy as jnp
from jax.experimental import pallas as pl


def kernel(node_feats, edge_feats, edge_index, W_e, b_e, W_a1, b_a1, W_a2, b_a2):
    raise NotImplementedError("write your pallas kernel here")


# speedup vs baseline: 2.4565x; 2.4565x over previous
import functools

import jax
import jax.numpy as jnp
from jax import lax
from jax.experimental import pallas as pl
from jax.experimental.pallas import tpu as pltpu
from jax.experimental.pallas import tpu_sc as plsc

N = 10000
E = 320000
D = 128

NC = 2
NS = 16
NW = NC * NS
EW = E // NW
C = 80
NCHUNK = EW // C
NP = 10240
NZ = NP // NS
ZC = 80
NZP = NZ // ZC


def _bent_half(z):
    return (jnp.sqrt(z * z + 1.0) - 1.0) * 0.25 + z * 0.5


def _sc_gather_body(node_hbm, src_hbm, out_hbm, idx_v, rows_v, sem):
    c = lax.axis_index("c")
    s = lax.axis_index("s")
    wid = s * NC + c
    base = wid * EW

    def chunk(j, carry):
        off = base + j * C
        pltpu.sync_copy(src_hbm.at[pl.ds(off, C)], idx_v)
        pltpu.async_copy(node_hbm.at[idx_v], rows_v, sem).wait()
        pltpu.sync_copy(rows_v, out_hbm.at[pl.ds(off, C)])
        return carry

    lax.fori_loop(0, NCHUNK, chunk, 0)


_sc_gather = pl.kernel(
    _sc_gather_body,
    out_type=jax.ShapeDtypeStruct((E, D), jnp.float32),
    mesh=plsc.VectorSubcoreMesh(core_axis_name="c", subcore_axis_name="s",
                                num_cores=NC, num_subcores=NS),
    scratch_types=[
        pltpu.VMEM((C,), jnp.int32),
        pltpu.VMEM((C, D), jnp.float32),
        pltpu.SemaphoreType.DMA,
    ],
)


def _sc_scatter_body(y_hbm, dst_hbm, part_hbm, idx_v, y_v, zero_v, acc_sp, sem):
    c = lax.axis_index("c")
    s = lax.axis_index("s")
    wid = s * NC + c
    base = wid * EW

    def zrow(i, carry):
        for j in range(D // 16):
            zero_v[i, pl.ds(j * 16, 16)] = jnp.zeros((16,), jnp.float32)
        return carry

    lax.fori_loop(0, ZC, zrow, 0)
    for k in range(NZP):
        pltpu.sync_copy(zero_v, acc_sp.at[pl.ds(s * NZ + k * ZC, ZC)])
    plsc.subcore_barrier()

    def chunk(j, carry):
        off = base + j * C
        pltpu.sync_copy(dst_hbm.at[pl.ds(off, C)], idx_v)
        pltpu.sync_copy(y_hbm.at[pl.ds(off, C)], y_v)
        pltpu.sync_copy(y_v, acc_sp.at[idx_v], add=True)
        return carry

    lax.fori_loop(0, NCHUNK, chunk, 0)
    plsc.subcore_barrier()

    pltpu.sync_copy(acc_sp.at[pl.ds(s * NZ, NZ)], part_hbm.at[c, pl.ds(s * NZ, NZ)])


_sc_scatter = pl.kernel(
    _sc_scatter_body,
    out_type=jax.ShapeDtypeStruct((NC, NP, D), jnp.float32),
    mesh=plsc.VectorSubcoreMesh(core_axis_name="c", subcore_axis_name="s",
                                num_cores=NC, num_subcores=NS),
    scratch_types=[
        pltpu.VMEM((C,), jnp.int32),
        pltpu.VMEM((C, D), jnp.float32),
        pltpu.VMEM((ZC, D), jnp.float32),
        pltpu.VMEM_SHARED((NP, D), jnp.float32),
        pltpu.SemaphoreType.DMA,
    ],
)


BE = 1600


def _tc_edge_mlp_body(g_ref, e_ref, w_ref, b_ref, y_ref):
    z = jnp.dot(g_ref[...] + e_ref[...], w_ref[...],
                preferred_element_type=jnp.float32) + b_ref[...]
    y_ref[...] = _bent_half(z)


def _tc_edge_mlp(g, edge_feats, w_e, b_e):
    return pl.pallas_call(
        _tc_edge_mlp_body,
        grid=(E // BE,),
        in_specs=[
            pl.BlockSpec((BE, D), lambda i: (i, 0)),
            pl.BlockSpec((BE, D), lambda i: (i, 0)),
            pl.BlockSpec((D, D), lambda i: (0, 0)),
            pl.BlockSpec((1, D), lambda i: (0, 0)),
        ],
        out_specs=pl.BlockSpec((BE, D), lambda i: (i, 0)),
        out_shape=jax.ShapeDtypeStruct((E, D), jnp.float32),
    )(g, edge_feats, w_e, b_e)


BN = 2000


def _tc_node_mlp_body(x_ref, p0_ref, p1_ref, w1_ref, b1_ref, w2_ref, b2_ref,
                      out_ref):
    x = x_ref[...] + (p0_ref[0] + p1_ref[0]) * 0.1
    z1 = jnp.dot(x * 0.5, w1_ref[...], preferred_element_type=jnp.float32) \
        + b1_ref[...]
    h = (jnp.sqrt(z1 * z1 + 1.0) - 1.0) * 0.5 + z1
    z2 = jnp.dot(h, w2_ref[...], preferred_element_type=jnp.float32) \
        + b2_ref[...]
    out_ref[...] = (jnp.sqrt(z2 * z2 + 1.0) - 1.0) * 0.5 + z2


def _tc_node_mlp(node_feats, parts, w_a1, b_a1, w_a2, b_a2):
    return pl.pallas_call(
        _tc_node_mlp_body,
        grid=(N // BN,),
        in_specs=[
            pl.BlockSpec((BN, D), lambda i: (i, 0)),
            pl.BlockSpec((1, BN, D), lambda i: (0, i, 0)),
            pl.BlockSpec((1, BN, D), lambda i: (1, i, 0)),
            pl.BlockSpec((D, D), lambda i: (0, 0)),
            pl.BlockSpec((1, D), lambda i: (0, 0)),
            pl.BlockSpec((D, D), lambda i: (0, 0)),
            pl.BlockSpec((1, D), lambda i: (0, 0)),
        ],
        out_specs=pl.BlockSpec((BN, D), lambda i: (i, 0)),
        out_shape=jax.ShapeDtypeStruct((N, D), jnp.float32),
    )(node_feats, parts, parts, w_a1, b_a1, w_a2, b_a2)


def kernel(node_feats, edge_feats, edge_index, W_e, b_e, W_a1, b_a1, W_a2, b_a2):
    src = edge_index[0].astype(jnp.int32)
    dst = edge_index[1].astype(jnp.int32)
    g = _sc_gather(node_feats, src)
    y = _tc_edge_mlp(g, edge_feats, W_e, b_e.reshape(1, D))
    parts = _sc_scatter(y, dst)
    x_out = _tc_node_mlp(node_feats, parts, W_a1, b_a1.reshape(1, D),
                         W_a2, b_a2.reshape(1, D))
    return (x_out, y)


# speedup vs baseline: 3.5458x; 1.4435x over previous
import functools

import jax
import jax.numpy as jnp
from jax import lax
from jax.experimental import pallas as pl
from jax.experimental.pallas import tpu as pltpu
from jax.experimental.pallas import tpu_sc as plsc

N = 10000
E = 320000
D = 128

NC = 2
NS = 16
NW = NC * NS
EW = E // NW
C = 80
NCHUNK = EW // C
NP = 10240
NZ = NP // NS
ZC = 80
NZP = NZ // ZC


def _bent_half(z):
    return (jnp.sqrt(z * z + 1.0) - 1.0) * 0.25 + z * 0.5


B = 400
K = B // C
NSC = EW // B
NPAIR = (NSC - 1) // 2


def _fire_gathers(node_hbm, idx_v, rows, j, sem):
    return [
        pltpu.async_copy(
            node_hbm.at[idx_v.at[pl.ds(j * B + i * C, C)]],
            rows.at[pl.ds(i * C, C)], sem)
        for i in range(K)
    ]


def _sc_gather_body(node_hbm, src_hbm, out_hbm, idx_v, rows0, rows1, gsem, wsem):
    c = lax.axis_index("c")
    s = lax.axis_index("s")
    wid = s * NC + c
    base = wid * EW
    rows = (rows0, rows1)

    pltpu.sync_copy(src_hbm.at[pl.ds(base, EW)], idx_v)
    for d in _fire_gathers(node_hbm, idx_v, rows0, 0, gsem):
        d.wait()

    def pair(jj, carry):
        j0 = jj * 2
        for b in range(2):
            j = j0 + b
            w = pltpu.async_copy(
                rows[b], out_hbm.at[pl.ds(base + j * B, B)], wsem)
            ds = _fire_gathers(node_hbm, idx_v, rows[1 - b], j + 1, gsem)
            for d in ds:
                d.wait()
            w.wait()
        return carry

    lax.fori_loop(0, NPAIR, pair, 0)
    pltpu.sync_copy(rows0, out_hbm.at[pl.ds(base + (NSC - 1) * B, B)])


_sc_gather = pl.kernel(
    _sc_gather_body,
    out_type=jax.ShapeDtypeStruct((E, D), jnp.float32),
    mesh=plsc.VectorSubcoreMesh(core_axis_name="c", subcore_axis_name="s",
                                num_cores=NC, num_subcores=NS),
    scratch_types=[
        pltpu.VMEM((EW,), jnp.int32),
        pltpu.VMEM((B, D), jnp.float32),
        pltpu.VMEM((B, D), jnp.float32),
        pltpu.SemaphoreType.DMA,
        pltpu.SemaphoreType.DMA,
    ],
)


NSCH = EW // C
NSPAIR = (NSCH - 1) // 2


def _sc_scatter_body(y_hbm, dst_hbm, part_hbm, y0, y1, i0, i1, acc_sp,
                     ysem, isem, ssem):
    c = lax.axis_index("c")
    s = lax.axis_index("s")
    wid = s * NC + c
    base = wid * EW
    ybuf = (y0, y1)
    ibank = (i0, i1)

    def zrow(i, carry):
        for j in range(D // 16):
            y0[i, pl.ds(j * 16, 16)] = jnp.zeros((16,), jnp.float32)
        return carry

    lax.fori_loop(0, C, zrow, 0)
    for k in range(NZ // C):
        pltpu.sync_copy(y0, acc_sp.at[pl.ds(s * NZ + k * C, C)])
    plsc.subcore_barrier()

    pltpu.sync_copy(y_hbm.at[pl.ds(base, C)], y0)
    pltpu.sync_copy(dst_hbm.at[pl.ds(base, C)], i0)

    def pair(jj, carry):
        j0 = jj * 2
        for b in range(2):
            j = j0 + b
            yd = pltpu.async_copy(y_hbm.at[pl.ds(base + (j + 1) * C, C)],
                                  ybuf[1 - b], ysem)
            idd = pltpu.async_copy(dst_hbm.at[pl.ds(base + (j + 1) * C, C)],
                                   ibank[1 - b], isem)
            pltpu.async_copy(ybuf[b], acc_sp.at[ibank[b]], ssem,
                             add=True).wait()
            yd.wait()
            idd.wait()
        return carry

    lax.fori_loop(0, NSPAIR, pair, 0)
    pltpu.sync_copy(y0, acc_sp.at[i0], add=True)
    plsc.subcore_barrier()

    pltpu.sync_copy(acc_sp.at[pl.ds(s * NZ, NZ)], part_hbm.at[c, pl.ds(s * NZ, NZ)])


_sc_scatter = pl.kernel(
    _sc_scatter_body,
    out_type=jax.ShapeDtypeStruct((NC, NP, D), jnp.float32),
    mesh=plsc.VectorSubcoreMesh(core_axis_name="c", subcore_axis_name="s",
                                num_cores=NC, num_subcores=NS),
    scratch_types=[
        pltpu.VMEM((C, D), jnp.float32),
        pltpu.VMEM((C, D), jnp.float32),
        pltpu.VMEM((C,), jnp.int32),
        pltpu.VMEM((C,), jnp.int32),
        pltpu.VMEM_SHARED((NP, D), jnp.float32),
        pltpu.SemaphoreType.DMA,
        pltpu.SemaphoreType.DMA,
        pltpu.SemaphoreType.DMA,
    ],
)


BE = 1600


def _tc_edge_mlp_body(g_ref, e_ref, w_ref, b_ref, y_ref):
    z = jnp.dot(g_ref[...] + e_ref[...], w_ref[...],
                preferred_element_type=jnp.float32) + b_ref[...]
    y_ref[...] = _bent_half(z)


def _tc_edge_mlp(g, edge_feats, w_e, b_e):
    return pl.pallas_call(
        _tc_edge_mlp_body,
        grid=(E // BE,),
        in_specs=[
            pl.BlockSpec((BE, D), lambda i: (i, 0)),
            pl.BlockSpec((BE, D), lambda i: (i, 0)),
            pl.BlockSpec((D, D), lambda i: (0, 0)),
            pl.BlockSpec((1, D), lambda i: (0, 0)),
        ],
        out_specs=pl.BlockSpec((BE, D), lambda i: (i, 0)),
        out_shape=jax.ShapeDtypeStruct((E, D), jnp.float32),
    )(g, edge_feats, w_e, b_e)


BN = 2000


def _tc_node_mlp_body(x_ref, p0_ref, p1_ref, w1_ref, b1_ref, w2_ref, b2_ref,
                      out_ref):
    x = x_ref[...] + (p0_ref[0] + p1_ref[0]) * 0.1
    z1 = jnp.dot(x * 0.5, w1_ref[...], preferred_element_type=jnp.float32) \
        + b1_ref[...]
    h = (jnp.sqrt(z1 * z1 + 1.0) - 1.0) * 0.5 + z1
    z2 = jnp.dot(h, w2_ref[...], preferred_element_type=jnp.float32) \
        + b2_ref[...]
    out_ref[...] = (jnp.sqrt(z2 * z2 + 1.0) - 1.0) * 0.5 + z2


def _tc_node_mlp(node_feats, parts, w_a1, b_a1, w_a2, b_a2):
    return pl.pallas_call(
        _tc_node_mlp_body,
        grid=(N // BN,),
        in_specs=[
            pl.BlockSpec((BN, D), lambda i: (i, 0)),
            pl.BlockSpec((1, BN, D), lambda i: (0, i, 0)),
            pl.BlockSpec((1, BN, D), lambda i: (1, i, 0)),
            pl.BlockSpec((D, D), lambda i: (0, 0)),
            pl.BlockSpec((1, D), lambda i: (0, 0)),
            pl.BlockSpec((D, D), lambda i: (0, 0)),
            pl.BlockSpec((1, D), lambda i: (0, 0)),
        ],
        out_specs=pl.BlockSpec((BN, D), lambda i: (i, 0)),
        out_shape=jax.ShapeDtypeStruct((N, D), jnp.float32),
    )(node_feats, parts, parts, w_a1, b_a1, w_a2, b_a2)


def kernel(node_feats, edge_feats, edge_index, W_e, b_e, W_a1, b_a1, W_a2, b_a2):
    src = edge_index[0].astype(jnp.int32)
    dst = edge_index[1].astype(jnp.int32)
    g = _sc_gather(node_feats, src)
    y = _tc_edge_mlp(g, edge_feats, W_e, b_e.reshape(1, D))
    parts = _sc_scatter(y, dst)
    x_out = _tc_node_mlp(node_feats, parts, W_a1, b_a1.reshape(1, D),
                         W_a2, b_a2.reshape(1, D))
    return (x_out, y)


# speedup vs baseline: 3.6536x; 1.0304x over previous
import functools

import jax
import jax.numpy as jnp
from jax import lax
from jax.experimental import pallas as pl
from jax.experimental.pallas import tpu as pltpu
from jax.experimental.pallas import tpu_sc as plsc

N = 10000
E = 320000
D = 128

NC = 2
NS = 16
NW = NC * NS
EW = E // NW
C = 80
NCHUNK = EW // C
NP = 10240
NZ = NP // NS
ZC = 80
NZP = NZ // ZC


def _bent_half(z):
    return (jnp.sqrt(z * z + 1.0) - 1.0) * 0.25 + z * 0.5


H = E // 2
EWH = H // NW
GB = 200
GC = 40
GK = GB // GC
GNSC = EWH // GB
GNPAIR = (GNSC - 1) // 2


def _make_sc_gather(src_off):
    def fire(node_hbm, idx_v, rows, j, sem):
        return [
            pltpu.async_copy(
                node_hbm.at[idx_v.at[pl.ds(j * GB + i * GC, GC)]],
                rows.at[pl.ds(i * GC, GC)], sem)
            for i in range(GK)
        ]

    def body(node_hbm, src_hbm, out_hbm, idx_v, rows0, rows1, gsem, wsem):
        c = lax.axis_index("c")
        s = lax.axis_index("s")
        wid = s * NC + c
        base = wid * EWH
        rows = (rows0, rows1)

        pltpu.sync_copy(src_hbm.at[pl.ds(src_off + base, EWH)], idx_v)
        for d in fire(node_hbm, idx_v, rows0, 0, gsem):
            d.wait()

        def pair(jj, carry):
            j0 = jj * 2
            for b in range(2):
                j = j0 + b
                w = pltpu.async_copy(
                    rows[b], out_hbm.at[pl.ds(base + j * GB, GB)], wsem)
                ds = fire(node_hbm, idx_v, rows[1 - b], j + 1, gsem)
                for d in ds:
                    d.wait()
                w.wait()
            return carry

        lax.fori_loop(0, GNPAIR, pair, 0)
        pltpu.sync_copy(rows0, out_hbm.at[pl.ds(base + (GNSC - 1) * GB, GB)])

    return pl.kernel(
        body,
        out_type=jax.ShapeDtypeStruct((H, D), jnp.float32),
        mesh=plsc.VectorSubcoreMesh(core_axis_name="c", subcore_axis_name="s",
                                    num_cores=NC, num_subcores=NS),
        scratch_types=[
            pltpu.VMEM((EWH,), jnp.int32),
            pltpu.VMEM((GB, D), jnp.float32),
            pltpu.VMEM((GB, D), jnp.float32),
            pltpu.SemaphoreType.DMA,
            pltpu.SemaphoreType.DMA,
        ],
    )


_sc_gather_lo = _make_sc_gather(0)
_sc_gather_hi = _make_sc_gather(H)


NSCH = EW // C
NSPAIR = (NSCH - 1) // 2


def _sc_scatter_body(y_hbm, dst_hbm, part_hbm, y0, y1, i0, i1, acc_sp,
                     ysem, isem, ssem):
    c = lax.axis_index("c")
    s = lax.axis_index("s")
    wid = s * NC + c
    base = wid * EW
    ybuf = (y0, y1)
    ibank = (i0, i1)

    def zrow(i, carry):
        for j in range(D // 16):
            y0[i, pl.ds(j * 16, 16)] = jnp.zeros((16,), jnp.float32)
        return carry

    lax.fori_loop(0, C, zrow, 0)
    for k in range(NZ // C):
        pltpu.sync_copy(y0, acc_sp.at[pl.ds(s * NZ + k * C, C)])
    plsc.subcore_barrier()

    pltpu.sync_copy(y_hbm.at[pl.ds(base, C)], y0)
    pltpu.sync_copy(dst_hbm.at[pl.ds(base, C)], i0)

    def pair(jj, carry):
        j0 = jj * 2
        for b in range(2):
            j = j0 + b
            yd = pltpu.async_copy(y_hbm.at[pl.ds(base + (j + 1) * C, C)],
                                  ybuf[1 - b], ysem)
            idd = pltpu.async_copy(dst_hbm.at[pl.ds(base + (j + 1) * C, C)],
                                   ibank[1 - b], isem)
            pltpu.async_copy(ybuf[b], acc_sp.at[ibank[b]], ssem,
                             add=True).wait()
            yd.wait()
            idd.wait()
        return carry

    lax.fori_loop(0, NSPAIR, pair, 0)
    pltpu.sync_copy(y0, acc_sp.at[i0], add=True)
    plsc.subcore_barrier()

    pltpu.sync_copy(acc_sp.at[pl.ds(s * NZ, NZ)], part_hbm.at[c, pl.ds(s * NZ, NZ)])


_sc_scatter = pl.kernel(
    _sc_scatter_body,
    out_type=jax.ShapeDtypeStruct((NC, NP, D), jnp.float32),
    mesh=plsc.VectorSubcoreMesh(core_axis_name="c", subcore_axis_name="s",
                                num_cores=NC, num_subcores=NS),
    scratch_types=[
        pltpu.VMEM((C, D), jnp.float32),
        pltpu.VMEM((C, D), jnp.float32),
        pltpu.VMEM((C,), jnp.int32),
        pltpu.VMEM((C,), jnp.int32),
        pltpu.VMEM_SHARED((NP, D), jnp.float32),
        pltpu.SemaphoreType.DMA,
        pltpu.SemaphoreType.DMA,
        pltpu.SemaphoreType.DMA,
    ],
)


BE = 1600
NBLK = H // BE


def _tc_edge_mlp_body(g_ref, e_ref, w_ref, b_ref, y_ref):
    z = jnp.dot(g_ref[...] + e_ref[...], w_ref[...],
                preferred_element_type=jnp.float32) + b_ref[...]
    y_ref[...] = _bent_half(z)


def _tc_edge_mlp_body_alias(g_ref, e_ref, w_ref, b_ref, _y_prev, y_ref):
    _tc_edge_mlp_body(g_ref, e_ref, w_ref, b_ref, y_ref)


def _tc_edge_mlp_half(g, edge_feats, w_e, b_e, blk_off, y_prev=None):
    args = [g, edge_feats, w_e, b_e]
    in_specs = [
        pl.BlockSpec((BE, D), lambda i: (i, 0)),
        pl.BlockSpec((BE, D), lambda i: (i + blk_off, 0)),
        pl.BlockSpec((D, D), lambda i: (0, 0)),
        pl.BlockSpec((1, D), lambda i: (0, 0)),
    ]
    kwargs = {}
    body = _tc_edge_mlp_body
    if y_prev is not None:
        args.append(y_prev)
        in_specs.append(pl.BlockSpec(memory_space=pl.ANY))
        kwargs["input_output_aliases"] = {4: 0}
        body = _tc_edge_mlp_body_alias
    return pl.pallas_call(
        body,
        grid=(NBLK,),
        in_specs=in_specs,
        out_specs=pl.BlockSpec((BE, D), lambda i: (i + blk_off, 0)),
        out_shape=jax.ShapeDtypeStruct((E, D), jnp.float32),
        **kwargs,
    )(*args)


BN = 2000


def _tc_node_mlp_body(x_ref, p0_ref, p1_ref, w1_ref, b1_ref, w2_ref, b2_ref,
                      out_ref):
    x = x_ref[...] + (p0_ref[0] + p1_ref[0]) * 0.1
    z1 = jnp.dot(x * 0.5, w1_ref[...], preferred_element_type=jnp.float32) \
        + b1_ref[...]
    h = (jnp.sqrt(z1 * z1 + 1.0) - 1.0) * 0.5 + z1
    z2 = jnp.dot(h, w2_ref[...], preferred_element_type=jnp.float32) \
        + b2_ref[...]
    out_ref[...] = (jnp.sqrt(z2 * z2 + 1.0) - 1.0) * 0.5 + z2


def _tc_node_mlp(node_feats, parts, w_a1, b_a1, w_a2, b_a2):
    return pl.pallas_call(
        _tc_node_mlp_body,
        grid=(N // BN,),
        in_specs=[
            pl.BlockSpec((BN, D), lambda i: (i, 0)),
            pl.BlockSpec((1, BN, D), lambda i: (0, i, 0)),
            pl.BlockSpec((1, BN, D), lambda i: (1, i, 0)),
            pl.BlockSpec((D, D), lambda i: (0, 0)),
            pl.BlockSpec((1, D), lambda i: (0, 0)),
            pl.BlockSpec((D, D), lambda i: (0, 0)),
            pl.BlockSpec((1, D), lambda i: (0, 0)),
        ],
        out_specs=pl.BlockSpec((BN, D), lambda i: (i, 0)),
        out_shape=jax.ShapeDtypeStruct((N, D), jnp.float32),
    )(node_feats, parts, parts, w_a1, b_a1, w_a2, b_a2)


def kernel(node_feats, edge_feats, edge_index, W_e, b_e, W_a1, b_a1, W_a2, b_a2):
    src = edge_index[0].astype(jnp.int32)
    dst = edge_index[1].astype(jnp.int32)
    g1 = _sc_gather_lo(node_feats, src)
    g2 = _sc_gather_hi(node_feats, src)
    y_lo = _tc_edge_mlp_half(g1, edge_feats, W_e, b_e.reshape(1, D), 0)
    y = _tc_edge_mlp_half(g2, edge_feats, W_e, b_e.reshape(1, D), NBLK,
                          y_prev=y_lo)
    parts = _sc_scatter(y, dst)
    x_out = _tc_node_mlp(node_feats, parts, W_a1, b_a1.reshape(1, D),
                         W_a2, b_a2.reshape(1, D))
    return (x_out, y)


# speedup vs baseline: 3.8607x; 1.0567x over previous
import functools

import jax
import jax.numpy as jnp
from jax import lax
from jax.experimental import pallas as pl
from jax.experimental.pallas import tpu as pltpu
from jax.experimental.pallas import tpu_sc as plsc

N = 10000
E = 320000
D = 128

NC = 2
NS = 16
NW = NC * NS
EW = E // NW
C = 80
NCHUNK = EW // C
NP = 10240
NZ = NP // NS
ZC = 80
NZP = NZ // ZC


def _bent_half(z):
    return (jnp.sqrt(z * z + 1.0) - 1.0) * 0.25 + z * 0.5


H = E // 2
EWH = H // NW
GB = 200
GC = 40
GK = GB // GC
GNSC = EWH // GB
GNPAIR = (GNSC - 1) // 2


def _make_sc_gather(src_off):
    def fire(node_hbm, idx_v, rows, j, sem):
        return [
            pltpu.async_copy(
                node_hbm.at[idx_v.at[pl.ds(j * GB + i * GC, GC)]],
                rows.at[pl.ds(i * GC, GC)], sem)
            for i in range(GK)
        ]

    def body(node_hbm, src_hbm, out_hbm, idx_v, rows0, rows1, gsem, wsem):
        c = lax.axis_index("c")
        s = lax.axis_index("s")
        wid = s * NC + c
        base = wid * EWH
        rows = (rows0, rows1)

        pltpu.sync_copy(src_hbm.at[pl.ds(src_off + base, EWH)], idx_v)
        for d in fire(node_hbm, idx_v, rows0, 0, gsem):
            d.wait()

        def pair(jj, carry):
            j0 = jj * 2
            for b in range(2):
                j = j0 + b
                w = pltpu.async_copy(
                    rows[b], out_hbm.at[pl.ds(base + j * GB, GB)], wsem)
                ds = fire(node_hbm, idx_v, rows[1 - b], j + 1, gsem)
                for d in ds:
                    d.wait()
                w.wait()
            return carry

        lax.fori_loop(0, GNPAIR, pair, 0)
        pltpu.sync_copy(rows0, out_hbm.at[pl.ds(base + (GNSC - 1) * GB, GB)])

    return pl.kernel(
        body,
        out_type=jax.ShapeDtypeStruct((H, D), jnp.float32),
        mesh=plsc.VectorSubcoreMesh(core_axis_name="c", subcore_axis_name="s",
                                    num_cores=NC, num_subcores=NS),
        scratch_types=[
            pltpu.VMEM((EWH,), jnp.int32),
            pltpu.VMEM((GB, D), jnp.float32),
            pltpu.VMEM((GB, D), jnp.float32),
            pltpu.SemaphoreType.DMA,
            pltpu.SemaphoreType.DMA,
        ],
    )


_sc_gather_lo = _make_sc_gather(0)
_sc_gather_hi = _make_sc_gather(H)


SCC = 128
NCHT = H // SCC
SPT = NCHT // NW
SREM = NCHT - SPT * NW
SPAIR = (SPT - 1) // 2


def _make_sc_scatter(dst_off):
    def body(y_hbm, dst_hbm, part_hbm, y0, y1, i0, i1, acc_sp,
             ysem, isem, ssem):
        c = lax.axis_index("c")
        s = lax.axis_index("s")
        wid = s * NC + c
        gbase = wid * SPT
        ybuf = (y0, y1)
        ibank = (i0, i1)

        def zrow(i, carry):
            for j in range(D // 16):
                y0[i, pl.ds(j * 16, 16)] = jnp.zeros((16,), jnp.float32)
            return carry

        lax.fori_loop(0, SCC, zrow, 0)
        for k in range(NZ // SCC):
            pltpu.sync_copy(y0, acc_sp.at[pl.ds(s * NZ + k * SCC, SCC)])
        plsc.subcore_barrier()

        pltpu.sync_copy(y_hbm.at[pl.ds(gbase * SCC, SCC)], y0)
        pltpu.sync_copy(dst_hbm.at[pl.ds(dst_off + gbase * SCC, SCC)], i0)

        def pair(jj, carry):
            j0 = jj * 2
            for b in range(2):
                j = j0 + b
                nxt = (gbase + j + 1) * SCC
                yd = pltpu.async_copy(y_hbm.at[pl.ds(nxt, SCC)],
                                      ybuf[1 - b], ysem)
                idd = pltpu.async_copy(dst_hbm.at[pl.ds(dst_off + nxt, SCC)],
                                       ibank[1 - b], isem)
                pltpu.async_copy(ybuf[b], acc_sp.at[ibank[b]], ssem,
                                 add=True).wait()
                yd.wait()
                idd.wait()
            return carry

        lax.fori_loop(0, SPAIR, pair, 0)
        pltpu.async_copy(y0, acc_sp.at[i0], ssem, add=True).wait()
        @pl.when(wid < SREM)
        def _rem():
            tail = (SPT * NW + wid) * SCC
            pltpu.sync_copy(y_hbm.at[pl.ds(tail, SCC)], y1)
            pltpu.sync_copy(dst_hbm.at[pl.ds(dst_off + tail, SCC)], i1)
            pltpu.async_copy(y1, acc_sp.at[i1], ssem, add=True).wait()

        plsc.subcore_barrier()

        pltpu.sync_copy(acc_sp.at[pl.ds(s * NZ, NZ)],
                        part_hbm.at[c, pl.ds(s * NZ, NZ)])

    return pl.kernel(
        body,
        out_type=jax.ShapeDtypeStruct((NC, NP, D), jnp.float32),
        mesh=plsc.VectorSubcoreMesh(core_axis_name="c", subcore_axis_name="s",
                                    num_cores=NC, num_subcores=NS),
        scratch_types=[
            pltpu.VMEM((SCC, D), jnp.float32),
            pltpu.VMEM((SCC, D), jnp.float32),
            pltpu.VMEM((SCC,), jnp.int32),
            pltpu.VMEM((SCC,), jnp.int32),
            pltpu.VMEM_SHARED((NP, D), jnp.float32),
            pltpu.SemaphoreType.DMA,
            pltpu.SemaphoreType.DMA,
            pltpu.SemaphoreType.DMA,
        ],
    )


_sc_scatter_lo = _make_sc_scatter(0)
_sc_scatter_hi = _make_sc_scatter(H)


BE = 1600
NBLK = H // BE


def _tc_edge_mlp_body(g_ref, e_ref, w_ref, b_ref, yh_ref, y_ref):
    z = jnp.dot(g_ref[...] + e_ref[...], w_ref[...],
                preferred_element_type=jnp.float32) + b_ref[...]
    v = _bent_half(z)
    yh_ref[...] = v
    y_ref[...] = v


def _tc_edge_mlp_body_alias(g_ref, e_ref, w_ref, b_ref, _y_prev, yh_ref, y_ref):
    _tc_edge_mlp_body(g_ref, e_ref, w_ref, b_ref, yh_ref, y_ref)


def _tc_edge_mlp_half(g, edge_feats, w_e, b_e, blk_off, y_prev=None):
    args = [g, edge_feats, w_e, b_e]
    in_specs = [
        pl.BlockSpec((BE, D), lambda i: (i, 0)),
        pl.BlockSpec((BE, D), lambda i: (i + blk_off, 0)),
        pl.BlockSpec((D, D), lambda i: (0, 0)),
        pl.BlockSpec((1, D), lambda i: (0, 0)),
    ]
    kwargs = {}
    body = _tc_edge_mlp_body
    if y_prev is not None:
        args.append(y_prev)
        in_specs.append(pl.BlockSpec(memory_space=pl.ANY))
        kwargs["input_output_aliases"] = {4: 1}
        body = _tc_edge_mlp_body_alias
    return pl.pallas_call(
        body,
        grid=(NBLK,),
        in_specs=in_specs,
        out_specs=[
            pl.BlockSpec((BE, D), lambda i: (i, 0)),
            pl.BlockSpec((BE, D), lambda i: (i + blk_off, 0)),
        ],
        out_shape=[
            jax.ShapeDtypeStruct((H, D), jnp.float32),
            jax.ShapeDtypeStruct((E, D), jnp.float32),
        ],
        **kwargs,
    )(*args)


BN = 2000


def _tc_node_mlp_body(x_ref, pa0_ref, pa1_ref, pb0_ref, pb1_ref,
                      w1_ref, b1_ref, w2_ref, b2_ref, out_ref):
    agg = pa0_ref[0] + pa1_ref[0] + pb0_ref[0] + pb1_ref[0]
    x = x_ref[...] + agg * 0.1
    z1 = jnp.dot(x * 0.5, w1_ref[...], preferred_element_type=jnp.float32) \
        + b1_ref[...]
    h = (jnp.sqrt(z1 * z1 + 1.0) - 1.0) * 0.5 + z1
    z2 = jnp.dot(h, w2_ref[...], preferred_element_type=jnp.float32) \
        + b2_ref[...]
    out_ref[...] = (jnp.sqrt(z2 * z2 + 1.0) - 1.0) * 0.5 + z2


def _tc_node_mlp(node_feats, parts_a, parts_b, w_a1, b_a1, w_a2, b_a2):
    return pl.pallas_call(
        _tc_node_mlp_body,
        grid=(N // BN,),
        in_specs=[
            pl.BlockSpec((BN, D), lambda i: (i, 0)),
            pl.BlockSpec((1, BN, D), lambda i: (0, i, 0)),
            pl.BlockSpec((1, BN, D), lambda i: (1, i, 0)),
            pl.BlockSpec((1, BN, D), lambda i: (0, i, 0)),
            pl.BlockSpec((1, BN, D), lambda i: (1, i, 0)),
            pl.BlockSpec((D, D), lambda i: (0, 0)),
            pl.BlockSpec((1, D), lambda i: (0, 0)),
            pl.BlockSpec((D, D), lambda i: (0, 0)),
            pl.BlockSpec((1, D), lambda i: (0, 0)),
        ],
        out_specs=pl.BlockSpec((BN, D), lambda i: (i, 0)),
        out_shape=jax.ShapeDtypeStruct((N, D), jnp.float32),
    )(node_feats, parts_a, parts_a, parts_b, parts_b, w_a1, b_a1, w_a2, b_a2)


def kernel(node_feats, edge_feats, edge_index, W_e, b_e, W_a1, b_a1, W_a2, b_a2):
    src = edge_index[0].astype(jnp.int32)
    dst = edge_index[1].astype(jnp.int32)
    g1 = _sc_gather_lo(node_feats, src)
    g2 = _sc_gather_hi(node_feats, src)
    yh1, yf1 = _tc_edge_mlp_half(g1, edge_feats, W_e, b_e.reshape(1, D), 0)
    yh2, y = _tc_edge_mlp_half(g2, edge_feats, W_e, b_e.reshape(1, D), NBLK,
                               y_prev=yf1)
    p1 = _sc_scatter_lo(yh1, dst)
    p2 = _sc_scatter_hi(yh2, dst)
    x_out = _tc_node_mlp(node_feats, p1, p2, W_a1, b_a1.reshape(1, D),
                         W_a2, b_a2.reshape(1, D))
    return (x_out, y)


# speedup vs baseline: 4.1841x; 1.0838x over previous
import functools

import jax
import jax.numpy as jnp
from jax import lax
from jax.experimental import pallas as pl
from jax.experimental.pallas import tpu as pltpu
from jax.experimental.pallas import tpu_sc as plsc

N = 10000
E = 320000
D = 128

NC = 2
NS = 16
NW = NC * NS
EW = E // NW
C = 80
NCHUNK = EW // C
NP = 10240
NZ = NP // NS
ZC = 80
NZP = NZ // ZC


def _bent_half(z):
    return (jnp.sqrt(z * z + 1.0) - 1.0) * 0.25 + z * 0.5


H = E // 2
EWH = H // NW
GB = 200
GC = 40
GK = GB // GC
GNSC = EWH // GB
GNPAIR = (GNSC - 1) // 2


def _make_sc_gather(src_off):
    def fire(node_hbm, idx_v, rows, j, sem):
        return [
            pltpu.async_copy(
                node_hbm.at[idx_v.at[pl.ds(j * GB + i * GC, GC)]],
                rows.at[pl.ds(i * GC, GC)], sem)
            for i in range(GK)
        ]

    def body(node_hbm, src_hbm, out_hbm, idx_v, rows0, rows1, gsem, wsem):
        c = lax.axis_index("c")
        s = lax.axis_index("s")
        wid = s * NC + c
        base = wid * EWH
        rows = (rows0, rows1)

        pltpu.sync_copy(src_hbm.at[pl.ds(src_off + base, EWH)], idx_v)
        for d in fire(node_hbm, idx_v, rows0, 0, gsem):
            d.wait()

        def pair(jj, carry):
            j0 = jj * 2
            for b in range(2):
                j = j0 + b
                w = pltpu.async_copy(
                    rows[b], out_hbm.at[pl.ds(base + j * GB, GB)], wsem)
                ds = fire(node_hbm, idx_v, rows[1 - b], j + 1, gsem)
                for d in ds:
                    d.wait()
                w.wait()
            return carry

        lax.fori_loop(0, GNPAIR, pair, 0)
        pltpu.sync_copy(rows0, out_hbm.at[pl.ds(base + (GNSC - 1) * GB, GB)])

    return pl.kernel(
        body,
        out_type=jax.ShapeDtypeStruct((H, D), jnp.float32),
        mesh=plsc.VectorSubcoreMesh(core_axis_name="c", subcore_axis_name="s",
                                    num_cores=NC, num_subcores=NS),
        scratch_types=[
            pltpu.VMEM((EWH,), jnp.int32),
            pltpu.VMEM((GB, D), jnp.float32),
            pltpu.VMEM((GB, D), jnp.float32),
            pltpu.SemaphoreType.DMA,
            pltpu.SemaphoreType.DMA,
        ],
    )


_sc_gather_lo = _make_sc_gather(0)
_sc_gather_hi = _make_sc_gather(H)


SCC = 128
NCHT = H // SCC
SPT = NCHT // NW
SREM = NCHT - SPT * NW
SPAIR = (SPT - 1) // 2


def _make_sc_scatter(dst_off):
    def body(y_hbm, dst_hbm, part_hbm, y0, y1, i0, i1, acc_sp,
             ysem, isem, ssem):
        c = lax.axis_index("c")
        s = lax.axis_index("s")
        wid = s * NC + c
        gbase = wid * SPT
        ybuf = (y0, y1)
        ibank = (i0, i1)

        def zrow(i, carry):
            for j in range(D // 16):
                y0[i, pl.ds(j * 16, 16)] = jnp.zeros((16,), jnp.float32)
            return carry

        lax.fori_loop(0, SCC, zrow, 0)
        for k in range(NZ // SCC):
            pltpu.sync_copy(y0, acc_sp.at[pl.ds(s * NZ + k * SCC, SCC)])
        plsc.subcore_barrier()

        pltpu.sync_copy(y_hbm.at[pl.ds(gbase * SCC, SCC)], y0)
        pltpu.sync_copy(dst_hbm.at[pl.ds(dst_off + gbase * SCC, SCC)], i0)

        def pair(jj, carry):
            j0 = jj * 2
            for b in range(2):
                j = j0 + b
                nxt = (gbase + j + 1) * SCC
                yd = pltpu.async_copy(y_hbm.at[pl.ds(nxt, SCC)],
                                      ybuf[1 - b], ysem)
                idd = pltpu.async_copy(dst_hbm.at[pl.ds(dst_off + nxt, SCC)],
                                       ibank[1 - b], isem)
                pltpu.async_copy(ybuf[b], acc_sp.at[ibank[b]], ssem,
                                 add=True).wait()
                yd.wait()
                idd.wait()
            return carry

        lax.fori_loop(0, SPAIR, pair, 0)
        pltpu.async_copy(y0, acc_sp.at[i0], ssem, add=True).wait()
        @pl.when(wid < SREM)
        def _rem():
            tail = (SPT * NW + wid) * SCC
            pltpu.sync_copy(y_hbm.at[pl.ds(tail, SCC)], y1)
            pltpu.sync_copy(dst_hbm.at[pl.ds(dst_off + tail, SCC)], i1)
            pltpu.async_copy(y1, acc_sp.at[i1], ssem, add=True).wait()

        plsc.subcore_barrier()

        pltpu.sync_copy(acc_sp.at[pl.ds(s * NZ, NZ)],
                        part_hbm.at[c, pl.ds(s * NZ, NZ)])

    return pl.kernel(
        body,
        out_type=jax.ShapeDtypeStruct((NC, NP, D), jnp.float32),
        mesh=plsc.VectorSubcoreMesh(core_axis_name="c", subcore_axis_name="s",
                                    num_cores=NC, num_subcores=NS),
        scratch_types=[
            pltpu.VMEM((SCC, D), jnp.float32),
            pltpu.VMEM((SCC, D), jnp.float32),
            pltpu.VMEM((SCC,), jnp.int32),
            pltpu.VMEM((SCC,), jnp.int32),
            pltpu.VMEM_SHARED((NP, D), jnp.float32),
            pltpu.SemaphoreType.DMA,
            pltpu.SemaphoreType.DMA,
            pltpu.SemaphoreType.DMA,
        ],
    )


_sc_scatter_lo = _make_sc_scatter(0)
_sc_scatter_hi = _make_sc_scatter(H)


BE = 4000
NBLK = H // BE


def _tc_edge_mlp_body(g_ref, e_ref, w_ref, b_ref, yh_ref, y_ref):
    z = jnp.dot(g_ref[...] + e_ref[...], w_ref[...],
                preferred_element_type=jnp.float32) + b_ref[...]
    v = _bent_half(z)
    yh_ref[...] = v
    y_ref[...] = v


def _tc_edge_mlp_body_alias(g_ref, e_ref, w_ref, b_ref, _y_prev, yh_ref, y_ref):
    _tc_edge_mlp_body(g_ref, e_ref, w_ref, b_ref, yh_ref, y_ref)


def _tc_edge_mlp_half(g, edge_feats, w_e, b_e, blk_off, y_prev=None):
    args = [g, edge_feats, w_e, b_e]
    in_specs = [
        pl.BlockSpec((BE, D), lambda i: (i, 0)),
        pl.BlockSpec((BE, D), lambda i: (i + blk_off, 0)),
        pl.BlockSpec((D, D), lambda i: (0, 0)),
        pl.BlockSpec((1, D), lambda i: (0, 0)),
    ]
    kwargs = {}
    body = _tc_edge_mlp_body
    if y_prev is not None:
        args.append(y_prev)
        in_specs.append(pl.BlockSpec(memory_space=pl.ANY))
        kwargs["input_output_aliases"] = {4: 1}
        body = _tc_edge_mlp_body_alias
    return pl.pallas_call(
        body,
        grid=(NBLK,),
        in_specs=in_specs,
        out_specs=[
            pl.BlockSpec((BE, D), lambda i: (i, 0)),
            pl.BlockSpec((BE, D), lambda i: (i + blk_off, 0)),
        ],
        out_shape=[
            jax.ShapeDtypeStruct((H, D), jnp.float32),
            jax.ShapeDtypeStruct((E, D), jnp.float32),
        ],
        **kwargs,
    )(*args)


BN = 2000


def _tc_node_mlp_body(x_ref, pa0_ref, pa1_ref, pb0_ref, pb1_ref,
                      w1_ref, b1_ref, w2_ref, b2_ref, out_ref):
    agg = pa0_ref[0] + pa1_ref[0] + pb0_ref[0] + pb1_ref[0]
    x = x_ref[...] + agg * 0.1
    z1 = jnp.dot(x * 0.5, w1_ref[...], preferred_element_type=jnp.float32) \
        + b1_ref[...]
    h = (jnp.sqrt(z1 * z1 + 1.0) - 1.0) * 0.5 + z1
    z2 = jnp.dot(h, w2_ref[...], preferred_element_type=jnp.float32) \
        + b2_ref[...]
    out_ref[...] = (jnp.sqrt(z2 * z2 + 1.0) - 1.0) * 0.5 + z2


def _tc_node_mlp(node_feats, parts_a, parts_b, w_a1, b_a1, w_a2, b_a2):
    return pl.pallas_call(
        _tc_node_mlp_body,
        grid=(N // BN,),
        in_specs=[
            pl.BlockSpec((BN, D), lambda i: (i, 0)),
            pl.BlockSpec((1, BN, D), lambda i: (0, i, 0)),
            pl.BlockSpec((1, BN, D), lambda i: (1, i, 0)),
            pl.BlockSpec((1, BN, D), lambda i: (0, i, 0)),
            pl.BlockSpec((1, BN, D), lambda i: (1, i, 0)),
            pl.BlockSpec((D, D), lambda i: (0, 0)),
            pl.BlockSpec((1, D), lambda i: (0, 0)),
            pl.BlockSpec((D, D), lambda i: (0, 0)),
            pl.BlockSpec((1, D), lambda i: (0, 0)),
        ],
        out_specs=pl.BlockSpec((BN, D), lambda i: (i, 0)),
        out_shape=jax.ShapeDtypeStruct((N, D), jnp.float32),
    )(node_feats, parts_a, parts_a, parts_b, parts_b, w_a1, b_a1, w_a2, b_a2)


def kernel(node_feats, edge_feats, edge_index, W_e, b_e, W_a1, b_a1, W_a2, b_a2):
    src = edge_index[0].astype(jnp.int32)
    dst = edge_index[1].astype(jnp.int32)
    g1 = _sc_gather_lo(node_feats, src)
    g2 = _sc_gather_hi(node_feats, src)
    yh1, yf1 = _tc_edge_mlp_half(g1, edge_feats, W_e, b_e.reshape(1, D), 0)
    yh2, y = _tc_edge_mlp_half(g2, edge_feats, W_e, b_e.reshape(1, D), NBLK,
                               y_prev=yf1)
    p1 = _sc_scatter_lo(yh1, dst)
    p2 = _sc_scatter_hi(yh2, dst)
    x_out = _tc_node_mlp(node_feats, p1, p2, W_a1, b_a1.reshape(1, D),
                         W_a2, b_a2.reshape(1, D))
    return (x_out, y)


# speedup vs baseline: 4.3500x; 1.0397x over previous
import functools

import jax
import jax.numpy as jnp
from jax import lax
from jax.experimental import pallas as pl
from jax.experimental.pallas import tpu as pltpu
from jax.experimental.pallas import tpu_sc as plsc

N = 10000
E = 320000
D = 128

NC = 2
NS = 16
NW = NC * NS
EW = E // NW
C = 80
NCHUNK = EW // C
NP = 10240
NZ = NP // NS
ZC = 80
NZP = NZ // ZC


def _bent_half(z):
    return (jnp.sqrt(z * z + 1.0) - 1.0) * 0.25 + z * 0.5


H = E // 2
EWH = H // NW
GB = 200
GC = 40
GK = GB // GC
GNSC = EWH // GB
GNPAIR = (GNSC - 1) // 2


def _make_sc_gather(src_off):
    def fire(node_hbm, idx_v, rows, j, sem):
        return [
            pltpu.async_copy(
                node_hbm.at[idx_v.at[pl.ds(j * GB + i * GC, GC)]],
                rows.at[pl.ds(i * GC, GC)], sem)
            for i in range(GK)
        ]

    def body(node_hbm, src_hbm, out_hbm, idx_v, rows0, rows1, gsem, wsem):
        c = lax.axis_index("c")
        s = lax.axis_index("s")
        wid = s * NC + c
        base = wid * EWH
        rows = (rows0, rows1)

        pltpu.sync_copy(src_hbm.at[pl.ds(src_off + base, EWH)], idx_v)
        for d in fire(node_hbm, idx_v, rows0, 0, gsem):
            d.wait()

        def pair(jj, carry):
            j0 = jj * 2
            for b in range(2):
                j = j0 + b
                w = pltpu.async_copy(
                    rows[b], out_hbm.at[pl.ds(base + j * GB, GB)], wsem)
                ds = fire(node_hbm, idx_v, rows[1 - b], j + 1, gsem)
                for d in ds:
                    d.wait()
                w.wait()
            return carry

        lax.fori_loop(0, GNPAIR, pair, 0)
        pltpu.sync_copy(rows0, out_hbm.at[pl.ds(base + (GNSC - 1) * GB, GB)])

    return pl.kernel(
        body,
        out_type=jax.ShapeDtypeStruct((H, D), jnp.float32),
        mesh=plsc.VectorSubcoreMesh(core_axis_name="c", subcore_axis_name="s",
                                    num_cores=NC, num_subcores=NS),
        scratch_types=[
            pltpu.VMEM((EWH,), jnp.int32),
            pltpu.VMEM((GB, D), jnp.float32),
            pltpu.VMEM((GB, D), jnp.float32),
            pltpu.SemaphoreType.DMA,
            pltpu.SemaphoreType.DMA,
        ],
    )


_sc_gather_lo = _make_sc_gather(0)
_sc_gather_hi = _make_sc_gather(H)


SCC = 128
NCHT = E // SCC
SPT = NCHT // NW
SREM = NCHT - SPT * NW
SPAIR = (SPT - 1) // 2


def _make_sc_scatter(dst_off):
    def body(y_hbm, dst_hbm, part_hbm, y0, y1, i0, i1, acc_sp,
             ysem, isem, ssem):
        c = lax.axis_index("c")
        s = lax.axis_index("s")
        wid = s * NC + c
        gbase = wid * SPT
        ybuf = (y0, y1)
        ibank = (i0, i1)

        def zrow(i, carry):
            for j in range(D // 16):
                y0[i, pl.ds(j * 16, 16)] = jnp.zeros((16,), jnp.float32)
            return carry

        lax.fori_loop(0, SCC, zrow, 0)
        for k in range(NZ // SCC):
            pltpu.sync_copy(y0, acc_sp.at[pl.ds(s * NZ + k * SCC, SCC)])
        plsc.subcore_barrier()

        pltpu.sync_copy(y_hbm.at[pl.ds(gbase * SCC, SCC)], y0)
        pltpu.sync_copy(dst_hbm.at[pl.ds(dst_off + gbase * SCC, SCC)], i0)

        def pair(jj, carry):
            j0 = jj * 2
            for b in range(2):
                j = j0 + b
                nxt = (gbase + j + 1) * SCC
                yd = pltpu.async_copy(y_hbm.at[pl.ds(nxt, SCC)],
                                      ybuf[1 - b], ysem)
                idd = pltpu.async_copy(dst_hbm.at[pl.ds(dst_off + nxt, SCC)],
                                       ibank[1 - b], isem)
                pltpu.async_copy(ybuf[b], acc_sp.at[ibank[b]], ssem,
                                 add=True).wait()
                yd.wait()
                idd.wait()
            return carry

        lax.fori_loop(0, SPAIR, pair, 0)
        if SPT % 2 == 0:
            nxt = (gbase + SPT - 1) * SCC
            yd = pltpu.async_copy(y_hbm.at[pl.ds(nxt, SCC)], y1, ysem)
            idd = pltpu.async_copy(dst_hbm.at[pl.ds(dst_off + nxt, SCC)],
                                   i1, isem)
            pltpu.async_copy(y0, acc_sp.at[i0], ssem, add=True).wait()
            yd.wait()
            idd.wait()
            pltpu.async_copy(y1, acc_sp.at[i1], ssem, add=True).wait()
        else:
            pltpu.async_copy(y0, acc_sp.at[i0], ssem, add=True).wait()
        @pl.when(wid < SREM)
        def _rem():
            tail = (SPT * NW + wid) * SCC
            pltpu.sync_copy(y_hbm.at[pl.ds(tail, SCC)], y1)
            pltpu.sync_copy(dst_hbm.at[pl.ds(dst_off + tail, SCC)], i1)
            pltpu.async_copy(y1, acc_sp.at[i1], ssem, add=True).wait()

        plsc.subcore_barrier()

        pltpu.sync_copy(acc_sp.at[pl.ds(s * NZ, NZ)],
                        part_hbm.at[c, pl.ds(s * NZ, NZ)])

    return pl.kernel(
        body,
        out_type=jax.ShapeDtypeStruct((NC, NP, D), jnp.float32),
        mesh=plsc.VectorSubcoreMesh(core_axis_name="c", subcore_axis_name="s",
                                    num_cores=NC, num_subcores=NS),
        scratch_types=[
            pltpu.VMEM((SCC, D), jnp.float32),
            pltpu.VMEM((SCC, D), jnp.float32),
            pltpu.VMEM((SCC,), jnp.int32),
            pltpu.VMEM((SCC,), jnp.int32),
            pltpu.VMEM_SHARED((NP, D), jnp.float32),
            pltpu.SemaphoreType.DMA,
            pltpu.SemaphoreType.DMA,
            pltpu.SemaphoreType.DMA,
        ],
    )


_sc_scatter = _make_sc_scatter(0)


BE = 4000
NBLK = H // BE


def _tc_edge_mlp_body(g_ref, e_ref, w_ref, b_ref, y_ref):
    z = jnp.dot(g_ref[...] + e_ref[...], w_ref[...],
                preferred_element_type=jnp.float32) + b_ref[...]
    y_ref[...] = _bent_half(z)


def _tc_edge_mlp_body_alias(g_ref, e_ref, w_ref, b_ref, _y_prev, y_ref):
    _tc_edge_mlp_body(g_ref, e_ref, w_ref, b_ref, y_ref)


def _tc_edge_mlp_half(g, edge_feats, w_e, b_e, blk_off, y_prev=None):
    args = [g, edge_feats, w_e, b_e]
    in_specs = [
        pl.BlockSpec((BE, D), lambda i: (i, 0)),
        pl.BlockSpec((BE, D), lambda i: (i + blk_off, 0)),
        pl.BlockSpec((D, D), lambda i: (0, 0)),
        pl.BlockSpec((1, D), lambda i: (0, 0)),
    ]
    kwargs = {}
    body = _tc_edge_mlp_body
    if y_prev is not None:
        args.append(y_prev)
        in_specs.append(pl.BlockSpec(memory_space=pl.ANY))
        kwargs["input_output_aliases"] = {4: 0}
        body = _tc_edge_mlp_body_alias
    return pl.pallas_call(
        body,
        grid=(NBLK,),
        in_specs=in_specs,
        out_specs=pl.BlockSpec((BE, D), lambda i: (i + blk_off, 0)),
        out_shape=jax.ShapeDtypeStruct((E, D), jnp.float32),
        **kwargs,
    )(*args)


BN = 2000


def _tc_node_mlp_body(x_ref, pa0_ref, pa1_ref,
                      w1_ref, b1_ref, w2_ref, b2_ref, out_ref):
    agg = pa0_ref[0] + pa1_ref[0]
    x = x_ref[...] + agg * 0.1
    z1 = jnp.dot(x * 0.5, w1_ref[...], preferred_element_type=jnp.float32) \
        + b1_ref[...]
    h = (jnp.sqrt(z1 * z1 + 1.0) - 1.0) * 0.5 + z1
    z2 = jnp.dot(h, w2_ref[...], preferred_element_type=jnp.float32) \
        + b2_ref[...]
    out_ref[...] = (jnp.sqrt(z2 * z2 + 1.0) - 1.0) * 0.5 + z2


def _tc_node_mlp(node_feats, parts_a, w_a1, b_a1, w_a2, b_a2):
    return pl.pallas_call(
        _tc_node_mlp_body,
        grid=(N // BN,),
        in_specs=[
            pl.BlockSpec((BN, D), lambda i: (i, 0)),
            pl.BlockSpec((1, BN, D), lambda i: (0, i, 0)),
            pl.BlockSpec((1, BN, D), lambda i: (1, i, 0)),
            pl.BlockSpec((D, D), lambda i: (0, 0)),
            pl.BlockSpec((1, D), lambda i: (0, 0)),
            pl.BlockSpec((D, D), lambda i: (0, 0)),
            pl.BlockSpec((1, D), lambda i: (0, 0)),
        ],
        out_specs=pl.BlockSpec((BN, D), lambda i: (i, 0)),
        out_shape=jax.ShapeDtypeStruct((N, D), jnp.float32),
    )(node_feats, parts_a, parts_a, w_a1, b_a1, w_a2, b_a2)


def kernel(node_feats, edge_feats, edge_index, W_e, b_e, W_a1, b_a1, W_a2, b_a2):
    src = edge_index[0].astype(jnp.int32)
    dst = edge_index[1].astype(jnp.int32)
    g1 = _sc_gather_lo(node_feats, src)
    g2 = _sc_gather_hi(node_feats, src)
    y_lo = _tc_edge_mlp_half(g1, edge_feats, W_e, b_e.reshape(1, D), 0)
    y = _tc_edge_mlp_half(g2, edge_feats, W_e, b_e.reshape(1, D), NBLK,
                          y_prev=y_lo)
    p1 = _sc_scatter(y, dst)
    x_out = _tc_node_mlp(node_feats, p1, W_a1, b_a1.reshape(1, D),
                         W_a2, b_a2.reshape(1, D))
    return (x_out, y)


# speedup vs baseline: 4.3501x; 1.0000x over previous
import functools

import jax
import jax.numpy as jnp
from jax import lax
from jax.experimental import pallas as pl
from jax.experimental.pallas import tpu as pltpu
from jax.experimental.pallas import tpu_sc as plsc

N = 10000
E = 320000
D = 128

NC = 2
NS = 16
NW = NC * NS
EW = E // NW
C = 80
NCHUNK = EW // C
NP = 10240
NZ = NP // NS
ZC = 80
NZP = NZ // ZC


def _bent_half(z):
    return (jnp.sqrt(z * z + 1.0) - 1.0) * 0.25 + z * 0.5


H = E // 2
EWH = H // NW
GCC = 128
GNCH = H // GCC
GPT = GNCH // NW
GREM = GNCH - GPT * NW
GPAIR = (GPT - 1) // 2
GIDX = GPT * GCC


def _make_sc_gather(src_off):
    def body(node_hbm, src_hbm, out_hbm, idx_v, rows0, rows1, gsem, wsem):
        c = lax.axis_index("c")
        s = lax.axis_index("s")
        wid = s * NC + c
        gbase = wid * GPT
        base = gbase * GCC
        rows = (rows0, rows1)

        pltpu.sync_copy(src_hbm.at[pl.ds(src_off + base, GIDX)],
                        idx_v.at[pl.ds(0, GIDX)])

        def fire(j, dst):
            return pltpu.async_copy(
                node_hbm.at[idx_v.at[pl.ds(j * GCC, GCC)]], dst, gsem)

        fire(0, rows0).wait()

        def pair(jj, carry):
            j0 = jj * 2
            for b in range(2):
                j = j0 + b
                w = pltpu.async_copy(
                    rows[b], out_hbm.at[pl.ds(base + j * GCC, GCC)], wsem)
                fire(j + 1, rows[1 - b]).wait()
                w.wait()
            return carry

        lax.fori_loop(0, GPAIR, pair, 0)
        pltpu.sync_copy(rows0, out_hbm.at[pl.ds(base + (GPT - 1) * GCC, GCC)])

        @pl.when(wid < GREM)
        def _rem():
            tail = (GPT * NW + wid) * GCC
            pltpu.sync_copy(src_hbm.at[pl.ds(src_off + tail, GCC)],
                            idx_v.at[pl.ds(GIDX, GCC)])
            pltpu.async_copy(
                node_hbm.at[idx_v.at[pl.ds(GIDX, GCC)]], rows1, gsem).wait()
            pltpu.sync_copy(rows1, out_hbm.at[pl.ds(tail, GCC)])

    return pl.kernel(
        body,
        out_type=jax.ShapeDtypeStruct((H, D), jnp.float32),
        mesh=plsc.VectorSubcoreMesh(core_axis_name="c", subcore_axis_name="s",
                                    num_cores=NC, num_subcores=NS),
        scratch_types=[
            pltpu.VMEM((GIDX + GCC,), jnp.int32),
            pltpu.VMEM((GCC, D), jnp.float32),
            pltpu.VMEM((GCC, D), jnp.float32),
            pltpu.SemaphoreType.DMA,
            pltpu.SemaphoreType.DMA,
        ],
    )


_sc_gather_lo = _make_sc_gather(0)
_sc_gather_hi = _make_sc_gather(H)


SCC = 128
NCHT = E // SCC
SPT = NCHT // NW
SREM = NCHT - SPT * NW
SPAIR = (SPT - 1) // 2


def _make_sc_scatter(dst_off):
    def body(y_hbm, dst_hbm, part_hbm, y0, y1, i0, i1, acc_sp,
             ysem, isem, ssem):
        c = lax.axis_index("c")
        s = lax.axis_index("s")
        wid = s * NC + c
        gbase = wid * SPT
        ybuf = (y0, y1)
        ibank = (i0, i1)

        def zrow(i, carry):
            for j in range(D // 16):
                y0[i, pl.ds(j * 16, 16)] = jnp.zeros((16,), jnp.float32)
            return carry

        lax.fori_loop(0, SCC, zrow, 0)
        for k in range(NZ // SCC):
            pltpu.sync_copy(y0, acc_sp.at[pl.ds(s * NZ + k * SCC, SCC)])
        plsc.subcore_barrier()

        pltpu.sync_copy(y_hbm.at[pl.ds(gbase * SCC, SCC)], y0)
        pltpu.sync_copy(dst_hbm.at[pl.ds(dst_off + gbase * SCC, SCC)], i0)

        def pair(jj, carry):
            j0 = jj * 2
            for b in range(2):
                j = j0 + b
                nxt = (gbase + j + 1) * SCC
                yd = pltpu.async_copy(y_hbm.at[pl.ds(nxt, SCC)],
                                      ybuf[1 - b], ysem)
                idd = pltpu.async_copy(dst_hbm.at[pl.ds(dst_off + nxt, SCC)],
                                       ibank[1 - b], isem)
                pltpu.async_copy(ybuf[b], acc_sp.at[ibank[b]], ssem,
                                 add=True).wait()
                yd.wait()
                idd.wait()
            return carry

        lax.fori_loop(0, SPAIR, pair, 0)
        if SPT % 2 == 0:
            nxt = (gbase + SPT - 1) * SCC
            yd = pltpu.async_copy(y_hbm.at[pl.ds(nxt, SCC)], y1, ysem)
            idd = pltpu.async_copy(dst_hbm.at[pl.ds(dst_off + nxt, SCC)],
                                   i1, isem)
            pltpu.async_copy(y0, acc_sp.at[i0], ssem, add=True).wait()
            yd.wait()
            idd.wait()
            pltpu.async_copy(y1, acc_sp.at[i1], ssem, add=True).wait()
        else:
            pltpu.async_copy(y0, acc_sp.at[i0], ssem, add=True).wait()
        @pl.when(wid < SREM)
        def _rem():
            tail = (SPT * NW + wid) * SCC
            pltpu.sync_copy(y_hbm.at[pl.ds(tail, SCC)], y1)
            pltpu.sync_copy(dst_hbm.at[pl.ds(dst_off + tail, SCC)], i1)
            pltpu.async_copy(y1, acc_sp.at[i1], ssem, add=True).wait()

        plsc.subcore_barrier()

        pltpu.sync_copy(acc_sp.at[pl.ds(s * NZ, NZ)],
                        part_hbm.at[c, pl.ds(s * NZ, NZ)])

    return pl.kernel(
        body,
        out_type=jax.ShapeDtypeStruct((NC, NP, D), jnp.float32),
        mesh=plsc.VectorSubcoreMesh(core_axis_name="c", subcore_axis_name="s",
                                    num_cores=NC, num_subcores=NS),
        scratch_types=[
            pltpu.VMEM((SCC, D), jnp.float32),
            pltpu.VMEM((SCC, D), jnp.float32),
            pltpu.VMEM((SCC,), jnp.int32),
            pltpu.VMEM((SCC,), jnp.int32),
            pltpu.VMEM_SHARED((NP, D), jnp.float32),
            pltpu.SemaphoreType.DMA,
            pltpu.SemaphoreType.DMA,
            pltpu.SemaphoreType.DMA,
        ],
    )


_sc_scatter = _make_sc_scatter(0)


BE = 8000
NBLK = H // BE


def _tc_edge_mlp_body(g_ref, e_ref, w_ref, b_ref, y_ref):
    z = jnp.dot(g_ref[...] + e_ref[...], w_ref[...],
                preferred_element_type=jnp.float32) + b_ref[...]
    y_ref[...] = _bent_half(z)


def _tc_edge_mlp_body_alias(g_ref, e_ref, w_ref, b_ref, _y_prev, y_ref):
    _tc_edge_mlp_body(g_ref, e_ref, w_ref, b_ref, y_ref)


def _tc_edge_mlp_half(g, edge_feats, w_e, b_e, blk_off, y_prev=None):
    args = [g, edge_feats, w_e, b_e]
    in_specs = [
        pl.BlockSpec((BE, D), lambda i: (i, 0)),
        pl.BlockSpec((BE, D), lambda i: (i + blk_off, 0)),
        pl.BlockSpec((D, D), lambda i: (0, 0)),
        pl.BlockSpec((1, D), lambda i: (0, 0)),
    ]
    kwargs = {}
    body = _tc_edge_mlp_body
    if y_prev is not None:
        args.append(y_prev)
        in_specs.append(pl.BlockSpec(memory_space=pl.ANY))
        kwargs["input_output_aliases"] = {4: 0}
        body = _tc_edge_mlp_body_alias
    return pl.pallas_call(
        body,
        grid=(NBLK,),
        in_specs=in_specs,
        out_specs=pl.BlockSpec((BE, D), lambda i: (i + blk_off, 0)),
        out_shape=jax.ShapeDtypeStruct((E, D), jnp.float32),
        **kwargs,
    )(*args)


BN = 2000


def _tc_node_mlp_body(x_ref, pa0_ref, pa1_ref,
                      w1_ref, b1_ref, w2_ref, b2_ref, out_ref):
    agg = pa0_ref[0] + pa1_ref[0]
    x = x_ref[...] + agg * 0.1
    z1 = jnp.dot(x * 0.5, w1_ref[...], preferred_element_type=jnp.float32) \
        + b1_ref[...]
    h = (jnp.sqrt(z1 * z1 + 1.0) - 1.0) * 0.5 + z1
    z2 = jnp.dot(h, w2_ref[...], preferred_element_type=jnp.float32) \
        + b2_ref[...]
    out_ref[...] = (jnp.sqrt(z2 * z2 + 1.0) - 1.0) * 0.5 + z2


def _tc_node_mlp(node_feats, parts_a, w_a1, b_a1, w_a2, b_a2):
    return pl.pallas_call(
        _tc_node_mlp_body,
        grid=(N // BN,),
        in_specs=[
            pl.BlockSpec((BN, D), lambda i: (i, 0)),
            pl.BlockSpec((1, BN, D), lambda i: (0, i, 0)),
            pl.BlockSpec((1, BN, D), lambda i: (1, i, 0)),
            pl.BlockSpec((D, D), lambda i: (0, 0)),
            pl.BlockSpec((1, D), lambda i: (0, 0)),
            pl.BlockSpec((D, D), lambda i: (0, 0)),
            pl.BlockSpec((1, D), lambda i: (0, 0)),
        ],
        out_specs=pl.BlockSpec((BN, D), lambda i: (i, 0)),
        out_shape=jax.ShapeDtypeStruct((N, D), jnp.float32),
    )(node_feats, parts_a, parts_a, w_a1, b_a1, w_a2, b_a2)


def kernel(node_feats, edge_feats, edge_index, W_e, b_e, W_a1, b_a1, W_a2, b_a2):
    src = edge_index[0].astype(jnp.int32)
    dst = edge_index[1].astype(jnp.int32)
    g1 = _sc_gather_lo(node_feats, src)
    g2 = _sc_gather_hi(node_feats, src)
    y_lo = _tc_edge_mlp_half(g1, edge_feats, W_e, b_e.reshape(1, D), 0)
    y = _tc_edge_mlp_half(g2, edge_feats, W_e, b_e.reshape(1, D), NBLK,
                          y_prev=y_lo)
    p1 = _sc_scatter(y, dst)
    x_out = _tc_node_mlp(node_feats, p1, W_a1, b_a1.reshape(1, D),
                         W_a2, b_a2.reshape(1, D))
    return (x_out, y)
